# baseline (speedup 1.0000x reference)
import functools

import jax
import jax.numpy as jnp
import numpy as np
from jax import lax
from jax.experimental import pallas as pl
from jax.experimental.pallas import tpu as pltpu
from jax.experimental.pallas import tpu_sc as plsc

NN = 10000
NE = 160000
DP = 128
DKV = 2 * DP
DN = 128
DSH = 16
ADIM = 86
RAD = 5.0
NB = 10
HID = 32
INV_SQRT_ATT = 1.0 / np.sqrt(ADIM)

NC, NS = 2, 16
NW = NC * NS
SUB = 32
NSUB = NE // SUB
NT_BASE = NSUB // NW
NT_REM = NSUB - NT_BASE * NW
ROWS_A = 624
ROWS_B = NN - (NS - 1) * ROWS_A
CGRP = 88

_mesh = functools.partial(
    plsc.VectorSubcoreMesh, core_axis_name="c", subcore_axis_name="s")


def _sc_gather_multi(tables, idxs):
    ntab = len(tables)
    widths = [t.shape[1] for t in tables]

    @functools.partial(
        pl.kernel,
        mesh=_mesh(),
        out_type=[jax.ShapeDtypeStruct((NE, w), jnp.float32)
                  for w in widths],
        scratch_types=(
            [pltpu.VMEM((SUB,), jnp.int32) for _ in range(ntab)]
            + [pltpu.VMEM((SUB, w), jnp.float32) for w in widths]
            + [pltpu.SemaphoreType.DMA]
        ),
    )
    def k(*refs):
        tab_h = refs[:ntab]
        idx_h = refs[ntab:2 * ntab]
        out_h = refs[2 * ntab:3 * ntab]
        ix = refs[3 * ntab:4 * ntab]
        rr = refs[4 * ntab:5 * ntab]
        sem = refs[5 * ntab]
        w = lax.axis_index("s") * NC + lax.axis_index("c")
        nt = NT_BASE + jnp.where(w < NT_REM, 1, 0)

        def body(t, carry):
            j = w + t * NW
            base = j * SUB
            for n in range(ntab):
                pltpu.sync_copy(idx_h[n].at[pl.ds(base, SUB)], ix[n])
            cps = [pltpu.async_copy(tab_h[n].at[ix[n]], rr[n], sem)
                   for n in range(ntab)]
            for cp in cps:
                cp.wait()
            for n in range(ntab):
                pltpu.sync_copy(rr[n], out_h[n].at[pl.ds(base, SUB)])
            return carry

        lax.fori_loop(0, nt, body, 0)

    return k(*tables, *idxs)


def _sc_edge_pass(qt, kvt, ekv, dsti, srci):
    @functools.partial(
        pl.kernel,
        mesh=_mesh(),
        out_type=jax.ShapeDtypeStruct((NC * NN, DN), jnp.float32),
        scratch_types=[
            pltpu.VMEM((SUB,), jnp.int32),
            pltpu.VMEM((SUB,), jnp.int32),
            pltpu.VMEM((SUB, DP), jnp.float32),
            pltpu.VMEM((SUB, DKV), jnp.float32),
            pltpu.VMEM((SUB, DKV), jnp.float32),
            pltpu.VMEM((SUB, DN), jnp.float32),
            pltpu.VMEM_SHARED((NN, DN), jnp.float32),
            pltpu.SemaphoreType.DMA,
        ],
    )
    def k(qt_h, kvt_h, ekv_h, dst_h, src_h, out_h,
          ixd, ixs, qb, kvb, eb, pb, acc, sem):
        c = lax.axis_index("c")
        s = lax.axis_index("s")
        w = s * NC + c
        nt = NT_BASE + jnp.where(w < NT_REM, 1, 0)
        rbase = s * ROWS_A

        def zrow(e, cz):
            for i in range(DN // 16):
                pb[e, pl.ds(i * 16, 16)] = jnp.zeros((16,), jnp.float32)
            return cz

        lax.fori_loop(0, SUB, zrow, 0)

        def zcp(i, cz):
            pltpu.sync_copy(pb, acc.at[pl.ds(rbase + i * SUB, SUB)])
            return cz

        lax.fori_loop(0, ROWS_A // SUB, zcp, 0)

        @pl.when(s < NS - 1)
        def _():
            pltpu.sync_copy(pb.at[pl.ds(0, ROWS_A - (ROWS_A // SUB) * SUB)],
                            acc.at[pl.ds(rbase + (ROWS_A // SUB) * SUB,
                                         ROWS_A - (ROWS_A // SUB) * SUB)])

        @pl.when(s == NS - 1)
        def _():
            pltpu.sync_copy(pb.at[pl.ds(0, ROWS_B - (ROWS_A // SUB) * SUB)],
                            acc.at[pl.ds(rbase + (ROWS_A // SUB) * SUB,
                                         ROWS_B - (ROWS_A // SUB) * SUB)])

        plsc.subcore_barrier()
        lane = jnp.arange(16, dtype=jnp.int32)

        def _hsum(a):
            for shf in (8, 4, 2, 1):
                a = a + a.at[lane ^ shf].get(mode="promise_in_bounds")
            return a

        def _bcast(a, l):
            return a.at[jnp.full((16,), l, jnp.int32)].get(
                mode="promise_in_bounds")

        def body(t, carry):
            j = w + t * NW
            base = j * SUB
            pltpu.sync_copy(dst_h.at[pl.ds(base, SUB)], ixd)
            pltpu.sync_copy(src_h.at[pl.ds(base, SUB)], ixs)
            cq = pltpu.async_copy(qt_h.at[ixd], qb, sem)
            ckv = pltpu.async_copy(kvt_h.at[ixs], kvb, sem)
            ce = pltpu.async_copy(ekv_h.at[pl.ds(base, SUB)], eb, sem)
            cq.wait()
            ckv.wait()
            ce.wait()

            def grp(g, c1):
                vec = jnp.zeros((16,), jnp.float32)
                for l in range(16):
                    e = g * 16 + l
                    a = jnp.zeros((16,), jnp.float32)
                    for i in range(6):
                        qv = qb[e, pl.ds(i * 16, 16)]
                        kv = kvb[e, pl.ds(i * 16, 16)]
                        ev = eb[e, pl.ds(i * 16, 16)]
                        a = a + qv * kv * ev
                    vec = jnp.where(lane == l, _hsum(a), vec)
                ex16 = jnp.exp(vec * INV_SQRT_ATT)
                for l in range(16):
                    e = g * 16 + l
                    exv = _bcast(ex16, l)
                    for i in range(DN // 16):
                        vv = kvb[e, pl.ds(DP + i * 16, 16)]
                        evv = eb[e, pl.ds(DP + i * 16, 16)]
                        pb[e, pl.ds(i * 16, 16)] = exv * vv * evv
                return c1

            lax.fori_loop(0, SUB // 16, grp, 0)

            pltpu.sync_copy(pb, acc.at[ixd], add=True)
            return carry

        lax.fori_loop(0, nt, body, 0)
        plsc.subcore_barrier()

        @pl.when(s < NS - 1)
        def _():
            pltpu.sync_copy(acc.at[pl.ds(rbase, ROWS_A)],
                            out_h.at[pl.ds(c * NN + rbase, ROWS_A)])

        @pl.when(s == NS - 1)
        def _():
            pltpu.sync_copy(acc.at[pl.ds(rbase, ROWS_B)],
                            out_h.at[pl.ds(c * NN + rbase, ROWS_B)])

    return k(qt, kvt, ekv, dsti, srci)


_BN = 1000
_BE = 2000


def _tc_node(h, wq, wkv):
    def body(h_ref, wq_ref, wkv_ref, q_ref, kv_ref):
        hb = h_ref[...]
        q_ref[...] = jnp.dot(hb, wq_ref[...],
                             preferred_element_type=jnp.float32)
        kv = jnp.dot(hb, wkv_ref[...], preferred_element_type=jnp.float32)
        col = lax.broadcasted_iota(jnp.int32, (_BN, DKV), 1)
        kv_ref[...] = jnp.where(col == DP + DN - 1, 1.0, kv)

    return pl.pallas_call(
        body,
        grid=(NN // _BN,),
        in_specs=[
            pl.BlockSpec((_BN, DN), lambda i: (i, 0)),
            pl.BlockSpec((DN, DP), lambda i: (0, 0)),
            pl.BlockSpec((DN, DKV), lambda i: (0, 0)),
        ],
        out_specs=[
            pl.BlockSpec((_BN, DP), lambda i: (i, 0)),
            pl.BlockSpec((_BN, DKV), lambda i: (i, 0)),
        ],
        out_shape=[
            jax.ShapeDtypeStruct((NN, DP), jnp.float32),
            jax.ShapeDtypeStruct((NN, DKV), jnp.float32),
        ],
    )(h, wq, wkv)


def _tc_geom(pos_src, pos_dst):
    def body(ps_ref, pd_ref, sh_ref, rbf_ref):
        rel = pd_ref[...] - ps_ref[...]
        xx = rel[:, 0:1]
        yy = rel[:, 1:2]
        zz = rel[:, 2:3]
        d = jnp.sqrt(xx * xx + yy * yy + zz * zz)
        inv = 1.0 / (d + 1e-9)
        x = xx * inv
        y = yy * inv
        z = zz * inv
        one = jnp.ones_like(x)
        sh = jnp.concatenate(
            [one, x, y, z,
             x * y, y * z, 0.5 * (3.0 * z * z - 1.0), x * z,
             0.5 * (x * x - y * y),
             y * (3.0 * x * x - y * y), x * y * z, y * (5.0 * z * z - 1.0),
             0.5 * z * (5.0 * z * z - 3.0), x * (5.0 * z * z - 1.0),
             z * (x * x - y * y), x * (x * x - 3.0 * y * y)], axis=1)
        sh_ref[...] = sh
        coli = lax.broadcasted_iota(jnp.int32, (_BE, DSH), 1)
        centers = coli.astype(jnp.float32) * (RAD / (NB - 1))
        rb = jnp.exp(-jnp.square((d - centers) / (RAD / NB)))
        mask = coli < NB
        rbf_ref[...] = jnp.where(mask, rb, 0.0)

    return pl.pallas_call(
        body,
        grid=(NE // _BE,),
        in_specs=[
            pl.BlockSpec((_BE, DP), lambda i: (i, 0)),
            pl.BlockSpec((_BE, DP), lambda i: (i, 0)),
        ],
        out_specs=[
            pl.BlockSpec((_BE, DSH), lambda i: (i, 0)),
            pl.BlockSpec((_BE, DSH), lambda i: (i, 0)),
        ],
        out_shape=[
            jax.ShapeDtypeStruct((NE, DSH), jnp.float32),
            jax.ShapeDtypeStruct((NE, DSH), jnp.float32),
        ],
    )(pos_src, pos_dst)


def _tc_ekv(sh, rbf, skp, svp, r1p, b1, rkp, rvp):
    def body(sh_ref, rbf_ref, sk_ref, sv_ref, r1_ref, b1_ref,
             rk_ref, rv_ref, out_ref):
        shb = sh_ref[...]
        r1 = jnp.maximum(
            jnp.dot(rbf_ref[...], r1_ref[...],
                    preferred_element_type=jnp.float32) + b1_ref[...], 0.0)
        ek = (jnp.dot(shb, sk_ref[...], preferred_element_type=jnp.float32)
              * jnp.dot(r1, rk_ref[...], preferred_element_type=jnp.float32))
        ev = (jnp.dot(shb, sv_ref[...], preferred_element_type=jnp.float32)
              * jnp.dot(r1, rv_ref[...], preferred_element_type=jnp.float32))
        ekv = jnp.concatenate([ek, ev], axis=1)
        col = lax.broadcasted_iota(jnp.int32, (_BE, DKV), 1)
        out_ref[...] = jnp.where(col == DP + DN - 1, 1.0, ekv)

    return pl.pallas_call(
        body,
        grid=(NE // _BE,),
        in_specs=[
            pl.BlockSpec((_BE, DSH), lambda i: (i, 0)),
            pl.BlockSpec((_BE, DSH), lambda i: (i, 0)),
            pl.BlockSpec((DSH, DP), lambda i: (0, 0)),
            pl.BlockSpec((DSH, DP), lambda i: (0, 0)),
            pl.BlockSpec((DSH, HID), lambda i: (0, 0)),
            pl.BlockSpec((1, HID), lambda i: (0, 0)),
            pl.BlockSpec((HID, DP), lambda i: (0, 0)),
            pl.BlockSpec((HID, DP), lambda i: (0, 0)),
        ],
        out_specs=pl.BlockSpec((_BE, DKV), lambda i: (i, 0)),
        out_shape=jax.ShapeDtypeStruct((NE, DKV), jnp.float32),
    )(sh, rbf, skp, svp, r1p, b1, rkp, rvp)


def _tc_finalize(s01, h, wsi, g, skip, dout):
    def body(s0_ref, s1_ref, h_ref, w_ref, g_ref, o_ref):
        num = s0_ref[...] + s1_ref[...]
        col = lax.broadcasted_iota(jnp.int32, (_BN, DN), 1)
        den = jnp.sum(jnp.where(col == (DN - 1), num, 0.0),
                      axis=1, keepdims=True) + 1e-9
        agg = jnp.where(col < dout, num, 0.0) / den
        hb = h_ref[...]
        out = agg + jnp.dot(hb, w_ref[...],
                            preferred_element_type=jnp.float32)
        if skip:
            out = out + hb
        ms = jnp.sum(out * out, axis=1, keepdims=True) / dout
        o_ref[...] = out / jnp.sqrt(ms + 1e-6) * g_ref[...]

    return pl.pallas_call(
        body,
        grid=(NN // _BN,),
        in_specs=[
            pl.BlockSpec((_BN, DN), lambda i: (i, 0)),
            pl.BlockSpec((_BN, DN), lambda i: (i + NN // _BN, 0)),
            pl.BlockSpec((_BN, DN), lambda i: (i, 0)),
            pl.BlockSpec((DN, DN), lambda i: (0, 0)),
            pl.BlockSpec((1, DN), lambda i: (0, 0)),
        ],
        out_specs=pl.BlockSpec((_BN, DN), lambda i: (i, 0)),
        out_shape=jax.ShapeDtypeStruct((NN, DN), jnp.float32),
    )(s01, s01, h, wsi, g)


def _pad2(a, r, c):
    return jnp.zeros((r, c), jnp.float32).at[:a.shape[0], :a.shape[1]].set(a)


def kernel(x, pos, params, edge_index):
    src = edge_index[0].astype(jnp.int32)
    dst = edge_index[1].astype(jnp.int32)

    pos_pad = _pad2(pos, NN, DP)
    x_pad = _pad2(x, NN, DN)

    pos_d, pos_s = _sc_gather_multi([pos_pad, pos_pad], [dst, src])
    sh, rbf = _tc_geom(pos_s, pos_d)

    h = x_pad
    douts = [86, 86, 86, 86, 86, 86, 40]
    skips = [False, True, True, True, True, True, False]
    for p, dout, skip in zip(params, douts, skips):
        din = p['Wq'].shape[0]
        wq = _pad2(p['Wq'], DN, DP)
        wkv = jnp.zeros((DN, DKV), jnp.float32)
        wkv = wkv.at[:din, :ADIM].set(p['Wk'])
        wkv = wkv.at[:din, DP:DP + dout].set(p['Wv'])
        skp = _pad2(p['Sk'], DSH, DP)
        svp = _pad2(p['Sv'], DSH, DP)
        r1p = _pad2(p['R1'], DSH, HID)
        b1 = p['b1'].reshape(1, HID)
        rkp = _pad2(p['Rk'], HID, DP)
        rvp = _pad2(p['Rv'], HID, DP)
        wsi = _pad2(p['Wsi'], DN, DN)
        g = _pad2(p['g'].reshape(1, -1), 1, DN)

        qt, kvt = _tc_node(h, wq, wkv)
        ekv = _tc_ekv(sh, rbf, skp, svp, r1p, b1, rkp, rvp)
        s01 = _sc_edge_pass(qt, kvt, ekv, dst, src)
        h = _tc_finalize(s01, h, wsi, g, skip, dout)

    return h[:, :40]


# speedup vs baseline: 1.0402x; 1.0402x over previous
import functools

import jax
import jax.numpy as jnp
import numpy as np
from jax import lax
from jax.experimental import pallas as pl
from jax.experimental.pallas import tpu as pltpu
from jax.experimental.pallas import tpu_sc as plsc

NN = 10000
NE = 160000
DP = 128
DKV = 2 * DP
DN = 128
DSH = 16
ADIM = 86
RAD = 5.0
NB = 10
HID = 32
INV_SQRT_ATT = 1.0 / np.sqrt(ADIM)

NC, NS = 2, 16
NW = NC * NS
SUB = 40
NSUB = NE // SUB
NT_BASE = NSUB // NW
NT_REM = NSUB - NT_BASE * NW
ROWS_A = 624
ROWS_B = NN - (NS - 1) * ROWS_A
CGRP = 88

_mesh = functools.partial(
    plsc.VectorSubcoreMesh, core_axis_name="c", subcore_axis_name="s")


def _sc_gather_multi(tables, idxs):
    ntab = len(tables)
    widths = [t.shape[1] for t in tables]

    @functools.partial(
        pl.kernel,
        mesh=_mesh(),
        out_type=[jax.ShapeDtypeStruct((NE, w), jnp.float32)
                  for w in widths],
        scratch_types=(
            [pltpu.VMEM((SUB,), jnp.int32) for _ in range(ntab)]
            + [pltpu.VMEM((SUB, w), jnp.float32) for w in widths]
            + [pltpu.SemaphoreType.DMA]
        ),
    )
    def k(*refs):
        tab_h = refs[:ntab]
        idx_h = refs[ntab:2 * ntab]
        out_h = refs[2 * ntab:3 * ntab]
        ix = refs[3 * ntab:4 * ntab]
        rr = refs[4 * ntab:5 * ntab]
        sem = refs[5 * ntab]
        w = lax.axis_index("s") * NC + lax.axis_index("c")
        nt = NT_BASE + jnp.where(w < NT_REM, 1, 0)

        def body(t, carry):
            j = w + t * NW
            base = j * SUB
            for n in range(ntab):
                pltpu.sync_copy(idx_h[n].at[pl.ds(base, SUB)], ix[n])
            cps = [pltpu.async_copy(tab_h[n].at[ix[n]], rr[n], sem)
                   for n in range(ntab)]
            for cp in cps:
                cp.wait()
            for n in range(ntab):
                pltpu.sync_copy(rr[n], out_h[n].at[pl.ds(base, SUB)])
            return carry

        lax.fori_loop(0, nt, body, 0)

    return k(*tables, *idxs)


def _sc_edge_pass(qt, kvt, ekv, dsti, srci):
    @functools.partial(
        pl.kernel,
        mesh=_mesh(),
        out_type=jax.ShapeDtypeStruct((NC * NN, DN), jnp.float32),
        scratch_types=[
            pltpu.VMEM((SUB,), jnp.int32),
            pltpu.VMEM((SUB,), jnp.int32),
            pltpu.VMEM((SUB, DP), jnp.float32),
            pltpu.VMEM((SUB, DKV), jnp.float32),
            pltpu.VMEM((SUB, DKV), jnp.float32),
            pltpu.VMEM((SUB, DN), jnp.float32),
            pltpu.VMEM_SHARED((NN, DN), jnp.float32),
            pltpu.SemaphoreType.DMA,
        ],
    )
    def k(qt_h, kvt_h, ekv_h, dst_h, src_h, out_h,
          ixd, ixs, qb, kvb, eb, pb, acc, sem):
        c = lax.axis_index("c")
        s = lax.axis_index("s")
        w = s * NC + c
        nt = NT_BASE + jnp.where(w < NT_REM, 1, 0)
        rbase = s * ROWS_A

        def zrow(e, cz):
            for i in range(DN // 16):
                pb[e, pl.ds(i * 16, 16)] = jnp.zeros((16,), jnp.float32)
            return cz

        lax.fori_loop(0, SUB, zrow, 0)

        def zcp(i, cz):
            pltpu.sync_copy(pb, acc.at[pl.ds(rbase + i * SUB, SUB)])
            return cz

        lax.fori_loop(0, ROWS_A // SUB, zcp, 0)

        @pl.when(s < NS - 1)
        def _():
            pltpu.sync_copy(pb.at[pl.ds(0, ROWS_A - (ROWS_A // SUB) * SUB)],
                            acc.at[pl.ds(rbase + (ROWS_A // SUB) * SUB,
                                         ROWS_A - (ROWS_A // SUB) * SUB)])

        @pl.when(s == NS - 1)
        def _():
            pltpu.sync_copy(pb.at[pl.ds(0, ROWS_B - (ROWS_A // SUB) * SUB)],
                            acc.at[pl.ds(rbase + (ROWS_A // SUB) * SUB,
                                         ROWS_B - (ROWS_A // SUB) * SUB)])

        plsc.subcore_barrier()
        lane = jnp.arange(16, dtype=jnp.int32)

        def _hsum(a):
            for shf in (8, 4, 2, 1):
                a = a + a.at[lane ^ shf].get(mode="promise_in_bounds")
            return a

        def body(t, carry):
            j = w + t * NW
            base = j * SUB
            pltpu.sync_copy(dst_h.at[pl.ds(base, SUB)], ixd)
            pltpu.sync_copy(src_h.at[pl.ds(base, SUB)], ixs)
            cq = pltpu.async_copy(qt_h.at[ixd], qb, sem)
            ckv = pltpu.async_copy(kvt_h.at[ixs], kvb, sem)
            ce = pltpu.async_copy(ekv_h.at[pl.ds(base, SUB)], eb, sem)
            cq.wait()
            ckv.wait()
            ce.wait()

            def edge(e, c1):
                a = jnp.zeros((16,), jnp.float32)
                for i in range(6):
                    qv = qb[e, pl.ds(i * 16, 16)]
                    kv = kvb[e, pl.ds(i * 16, 16)]
                    ev = eb[e, pl.ds(i * 16, 16)]
                    a = a + qv * kv * ev
                exv = jnp.exp(_hsum(a) * INV_SQRT_ATT)
                for i in range(DN // 16):
                    vv = kvb[e, pl.ds(DP + i * 16, 16)]
                    evv = eb[e, pl.ds(DP + i * 16, 16)]
                    pb[e, pl.ds(i * 16, 16)] = exv * vv * evv
                return c1

            lax.fori_loop(0, SUB, edge, 0)

            pltpu.sync_copy(pb, acc.at[ixd], add=True)
            return carry

        lax.fori_loop(0, nt, body, 0)
        plsc.subcore_barrier()

        @pl.when(s < NS - 1)
        def _():
            pltpu.sync_copy(acc.at[pl.ds(rbase, ROWS_A)],
                            out_h.at[pl.ds(c * NN + rbase, ROWS_A)])

        @pl.when(s == NS - 1)
        def _():
            pltpu.sync_copy(acc.at[pl.ds(rbase, ROWS_B)],
                            out_h.at[pl.ds(c * NN + rbase, ROWS_B)])

    return k(qt, kvt, ekv, dsti, srci)


_BN = 1000
_BE = 2000


def _tc_node(h, wq, wkv):
    def body(h_ref, wq_ref, wkv_ref, q_ref, kv_ref):
        hb = h_ref[...]
        q_ref[...] = jnp.dot(hb, wq_ref[...],
                             preferred_element_type=jnp.float32)
        kv = jnp.dot(hb, wkv_ref[...], preferred_element_type=jnp.float32)
        col = lax.broadcasted_iota(jnp.int32, (_BN, DKV), 1)
        kv_ref[...] = jnp.where(col == DP + DN - 1, 1.0, kv)

    return pl.pallas_call(
        body,
        grid=(NN // _BN,),
        in_specs=[
            pl.BlockSpec((_BN, DN), lambda i: (i, 0)),
            pl.BlockSpec((DN, DP), lambda i: (0, 0)),
            pl.BlockSpec((DN, DKV), lambda i: (0, 0)),
        ],
        out_specs=[
            pl.BlockSpec((_BN, DP), lambda i: (i, 0)),
            pl.BlockSpec((_BN, DKV), lambda i: (i, 0)),
        ],
        out_shape=[
            jax.ShapeDtypeStruct((NN, DP), jnp.float32),
            jax.ShapeDtypeStruct((NN, DKV), jnp.float32),
        ],
    )(h, wq, wkv)


def _tc_geom(pos_src, pos_dst):
    def body(ps_ref, pd_ref, sh_ref, rbf_ref):
        rel = pd_ref[...] - ps_ref[...]
        xx = rel[:, 0:1]
        yy = rel[:, 1:2]
        zz = rel[:, 2:3]
        d = jnp.sqrt(xx * xx + yy * yy + zz * zz)
        inv = 1.0 / (d + 1e-9)
        x = xx * inv
        y = yy * inv
        z = zz * inv
        one = jnp.ones_like(x)
        sh = jnp.concatenate(
            [one, x, y, z,
             x * y, y * z, 0.5 * (3.0 * z * z - 1.0), x * z,
             0.5 * (x * x - y * y),
             y * (3.0 * x * x - y * y), x * y * z, y * (5.0 * z * z - 1.0),
             0.5 * z * (5.0 * z * z - 3.0), x * (5.0 * z * z - 1.0),
             z * (x * x - y * y), x * (x * x - 3.0 * y * y)], axis=1)
        sh_ref[...] = sh
        coli = lax.broadcasted_iota(jnp.int32, (_BE, DSH), 1)
        centers = coli.astype(jnp.float32) * (RAD / (NB - 1))
        rb = jnp.exp(-jnp.square((d - centers) / (RAD / NB)))
        mask = coli < NB
        rbf_ref[...] = jnp.where(mask, rb, 0.0)

    return pl.pallas_call(
        body,
        grid=(NE // _BE,),
        in_specs=[
            pl.BlockSpec((_BE, DP), lambda i: (i, 0)),
            pl.BlockSpec((_BE, DP), lambda i: (i, 0)),
        ],
        out_specs=[
            pl.BlockSpec((_BE, DSH), lambda i: (i, 0)),
            pl.BlockSpec((_BE, DSH), lambda i: (i, 0)),
        ],
        out_shape=[
            jax.ShapeDtypeStruct((NE, DSH), jnp.float32),
            jax.ShapeDtypeStruct((NE, DSH), jnp.float32),
        ],
    )(pos_src, pos_dst)


def _tc_ekv(sh, rbf, skp, svp, r1p, b1, rkp, rvp):
    def body(sh_ref, rbf_ref, sk_ref, sv_ref, r1_ref, b1_ref,
             rk_ref, rv_ref, out_ref):
        shb = sh_ref[...]
        r1 = jnp.maximum(
            jnp.dot(rbf_ref[...], r1_ref[...],
                    preferred_element_type=jnp.float32) + b1_ref[...], 0.0)
        ek = (jnp.dot(shb, sk_ref[...], preferred_element_type=jnp.float32)
              * jnp.dot(r1, rk_ref[...], preferred_element_type=jnp.float32))
        ev = (jnp.dot(shb, sv_ref[...], preferred_element_type=jnp.float32)
              * jnp.dot(r1, rv_ref[...], preferred_element_type=jnp.float32))
        ekv = jnp.concatenate([ek, ev], axis=1)
        col = lax.broadcasted_iota(jnp.int32, (_BE, DKV), 1)
        out_ref[...] = jnp.where(col == DP + DN - 1, 1.0, ekv)

    return pl.pallas_call(
        body,
        grid=(NE // _BE,),
        in_specs=[
            pl.BlockSpec((_BE, DSH), lambda i: (i, 0)),
            pl.BlockSpec((_BE, DSH), lambda i: (i, 0)),
            pl.BlockSpec((DSH, DP), lambda i: (0, 0)),
            pl.BlockSpec((DSH, DP), lambda i: (0, 0)),
            pl.BlockSpec((DSH, HID), lambda i: (0, 0)),
            pl.BlockSpec((1, HID), lambda i: (0, 0)),
            pl.BlockSpec((HID, DP), lambda i: (0, 0)),
            pl.BlockSpec((HID, DP), lambda i: (0, 0)),
        ],
        out_specs=pl.BlockSpec((_BE, DKV), lambda i: (i, 0)),
        out_shape=jax.ShapeDtypeStruct((NE, DKV), jnp.float32),
    )(sh, rbf, skp, svp, r1p, b1, rkp, rvp)


def _tc_finalize(s01, h, wsi, g, skip, dout):
    def body(s0_ref, s1_ref, h_ref, w_ref, g_ref, o_ref):
        num = s0_ref[...] + s1_ref[...]
        col = lax.broadcasted_iota(jnp.int32, (_BN, DN), 1)
        den = jnp.sum(jnp.where(col == (DN - 1), num, 0.0),
                      axis=1, keepdims=True) + 1e-9
        agg = jnp.where(col < dout, num, 0.0) / den
        hb = h_ref[...]
        out = agg + jnp.dot(hb, w_ref[...],
                            preferred_element_type=jnp.float32)
        if skip:
            out = out + hb
        ms = jnp.sum(out * out, axis=1, keepdims=True) / dout
        o_ref[...] = out / jnp.sqrt(ms + 1e-6) * g_ref[...]

    return pl.pallas_call(
        body,
        grid=(NN // _BN,),
        in_specs=[
            pl.BlockSpec((_BN, DN), lambda i: (i, 0)),
            pl.BlockSpec((_BN, DN), lambda i: (i + NN // _BN, 0)),
            pl.BlockSpec((_BN, DN), lambda i: (i, 0)),
            pl.BlockSpec((DN, DN), lambda i: (0, 0)),
            pl.BlockSpec((1, DN), lambda i: (0, 0)),
        ],
        out_specs=pl.BlockSpec((_BN, DN), lambda i: (i, 0)),
        out_shape=jax.ShapeDtypeStruct((NN, DN), jnp.float32),
    )(s01, s01, h, wsi, g)


def _pad2(a, r, c):
    return jnp.zeros((r, c), jnp.float32).at[:a.shape[0], :a.shape[1]].set(a)


def kernel(x, pos, params, edge_index):
    src = edge_index[0].astype(jnp.int32)
    dst = edge_index[1].astype(jnp.int32)

    pos_pad = _pad2(pos, NN, DP)
    x_pad = _pad2(x, NN, DN)

    pos_d, pos_s = _sc_gather_multi([pos_pad, pos_pad], [dst, src])
    sh, rbf = _tc_geom(pos_s, pos_d)

    h = x_pad
    douts = [86, 86, 86, 86, 86, 86, 40]
    skips = [False, True, True, True, True, True, False]
    for p, dout, skip in zip(params, douts, skips):
        din = p['Wq'].shape[0]
        wq = _pad2(p['Wq'], DN, DP)
        wkv = jnp.zeros((DN, DKV), jnp.float32)
        wkv = wkv.at[:din, :ADIM].set(p['Wk'])
        wkv = wkv.at[:din, DP:DP + dout].set(p['Wv'])
        skp = _pad2(p['Sk'], DSH, DP)
        svp = _pad2(p['Sv'], DSH, DP)
        r1p = _pad2(p['R1'], DSH, HID)
        b1 = p['b1'].reshape(1, HID)
        rkp = _pad2(p['Rk'], HID, DP)
        rvp = _pad2(p['Rv'], HID, DP)
        wsi = _pad2(p['Wsi'], DN, DN)
        g = _pad2(p['g'].reshape(1, -1), 1, DN)

        qt, kvt = _tc_node(h, wq, wkv)
        ekv = _tc_ekv(sh, rbf, skp, svp, r1p, b1, rkp, rvp)
        s01 = _sc_edge_pass(qt, kvt, ekv, dst, src)
        h = _tc_finalize(s01, h, wsi, g, skip, dout)

    return h[:, :40]


# speedup vs baseline: 1.3148x; 1.2639x over previous
import functools

import jax
import jax.numpy as jnp
import numpy as np
from jax import lax
from jax.experimental import pallas as pl
from jax.experimental.pallas import tpu as pltpu
from jax.experimental.pallas import tpu_sc as plsc

NN = 10000
NE = 160000
DP = 128
DKV = 2 * DP
DN = 128
DSH = 16
ADIM = 86
RAD = 5.0
NB = 10
HID = 32
INV_SQRT_ATT = 1.0 / np.sqrt(ADIM)

NC, NS = 2, 16
NW = NC * NS
SUB = 32
NSUB = NE // SUB
NT_BASE = NSUB // NW
NT_REM = NSUB - NT_BASE * NW
ROWS_A = 624
ROWS_B = NN - (NS - 1) * ROWS_A
CGRP = 88

_mesh = functools.partial(
    plsc.VectorSubcoreMesh, core_axis_name="c", subcore_axis_name="s")


def _sc_gather_multi(tables, idxs):
    ntab = len(tables)
    widths = [t.shape[1] for t in tables]

    @functools.partial(
        pl.kernel,
        mesh=_mesh(),
        out_type=[jax.ShapeDtypeStruct((NE, w), jnp.float32)
                  for w in widths],
        scratch_types=(
            [pltpu.VMEM((SUB,), jnp.int32) for _ in range(ntab)]
            + [pltpu.VMEM((SUB, w), jnp.float32) for w in widths]
            + [pltpu.SemaphoreType.DMA]
        ),
    )
    def k(*refs):
        tab_h = refs[:ntab]
        idx_h = refs[ntab:2 * ntab]
        out_h = refs[2 * ntab:3 * ntab]
        ix = refs[3 * ntab:4 * ntab]
        rr = refs[4 * ntab:5 * ntab]
        sem = refs[5 * ntab]
        w = lax.axis_index("s") * NC + lax.axis_index("c")
        nt = NT_BASE + jnp.where(w < NT_REM, 1, 0)

        def body(t, carry):
            j = w + t * NW
            base = j * SUB
            for n in range(ntab):
                pltpu.sync_copy(idx_h[n].at[pl.ds(base, SUB)], ix[n])
            cps = [pltpu.async_copy(tab_h[n].at[ix[n]], rr[n], sem)
                   for n in range(ntab)]
            for cp in cps:
                cp.wait()
            for n in range(ntab):
                pltpu.sync_copy(rr[n], out_h[n].at[pl.ds(base, SUB)])
            return carry

        lax.fori_loop(0, nt, body, 0)

    return k(*tables, *idxs)


def _sc_edge_pass(qt, kvt, ekv, dsti, srci):
    @functools.partial(
        pl.kernel,
        mesh=_mesh(),
        out_type=jax.ShapeDtypeStruct((NC * NN, DN), jnp.float32),
        scratch_types=[
            pltpu.VMEM((SUB,), jnp.int32),
            pltpu.VMEM((SUB,), jnp.int32),
            pltpu.VMEM((SUB,), jnp.int32),
            pltpu.VMEM((SUB,), jnp.int32),
            pltpu.VMEM((SUB,), jnp.int32),
            pltpu.VMEM((SUB,), jnp.int32),
            pltpu.VMEM((SUB, DP), jnp.float32),
            pltpu.VMEM((SUB, DP), jnp.float32),
            pltpu.VMEM((SUB, DKV), jnp.float32),
            pltpu.VMEM((SUB, DKV), jnp.float32),
            pltpu.VMEM((SUB, DKV), jnp.float32),
            pltpu.VMEM((SUB, DKV), jnp.float32),
            pltpu.VMEM((SUB, DN), jnp.float32),
            pltpu.VMEM((SUB, DN), jnp.float32),
            pltpu.VMEM_SHARED((NN, DN), jnp.float32),
            pltpu.SemaphoreType.DMA,
            pltpu.SemaphoreType.DMA,
        ],
    )
    def k(qt_h, kvt_h, ekv_h, dst_h, src_h, out_h,
          ixd0, ixd1, ixs0, ixs1, ixc0, ixc1,
          qb0, qb1, kvb0, kvb1, eb0, eb1, pb0, pb1,
          acc, semg, sems):
        ixd = (ixd0, ixd1)
        ixs = (ixs0, ixs1)
        ixc = (ixc0, ixc1)
        qb = (qb0, qb1)
        kvb = (kvb0, kvb1)
        eb = (eb0, eb1)
        pbs = (pb0, pb1)
        pb = pb0
        c = lax.axis_index("c")
        s = lax.axis_index("s")
        w = s * NC + c
        rbase = s * ROWS_A

        def zrow(e, cz):
            for i in range(DN // 16):
                pb[e, pl.ds(i * 16, 16)] = jnp.zeros((16,), jnp.float32)
            return cz

        lax.fori_loop(0, SUB, zrow, 0)

        def zcp(i, cz):
            pltpu.sync_copy(pb, acc.at[pl.ds(rbase + i * SUB, SUB)])
            return cz

        lax.fori_loop(0, ROWS_A // SUB, zcp, 0)

        @pl.when(s < NS - 1)
        def _():
            pltpu.sync_copy(pb.at[pl.ds(0, ROWS_A - (ROWS_A // SUB) * SUB)],
                            acc.at[pl.ds(rbase + (ROWS_A // SUB) * SUB,
                                         ROWS_A - (ROWS_A // SUB) * SUB)])

        @pl.when(s == NS - 1)
        def _():
            pltpu.sync_copy(pb.at[pl.ds(0, ROWS_B - (ROWS_A // SUB) * SUB)],
                            acc.at[pl.ds(rbase + (ROWS_A // SUB) * SUB,
                                         ROWS_B - (ROWS_A // SUB) * SUB)])

        plsc.subcore_barrier()
        lane = jnp.arange(16, dtype=jnp.int32)

        def _hsum(a):
            for shf in (8, 4, 2, 1):
                a = a + a.at[lane ^ shf].get(mode="promise_in_bounds")
            return a

        def _compute(q_b, kv_b, e_b, p_b):
            def edge(e, c1):
                a = jnp.zeros((16,), jnp.float32)
                for i in range(6):
                    qv = q_b[e, pl.ds(i * 16, 16)]
                    kv = kv_b[e, pl.ds(i * 16, 16)]
                    ev = e_b[e, pl.ds(i * 16, 16)]
                    a = a + qv * kv * ev
                exv = jnp.exp(_hsum(a) * INV_SQRT_ATT)
                for i in range(DN // 16):
                    vv = kv_b[e, pl.ds(DP + i * 16, 16)]
                    evv = e_b[e, pl.ds(DP + i * 16, 16)]
                    p_b[e, pl.ds(i * 16, 16)] = exv * vv * evv
                return c1

            lax.fori_loop(0, SUB, edge, 0)

        def _load_idx(base, b):
            pltpu.sync_copy(dst_h.at[pl.ds(base, SUB)], ixd[b])
            pltpu.sync_copy(src_h.at[pl.ds(base, SUB)], ixs[b])

        def _fire(base, b):
            pltpu.async_copy(qt_h.at[ixd[b]], qb[b], semg)
            pltpu.async_copy(kvt_h.at[ixs[b]], kvb[b], semg)
            pltpu.async_copy(ekv_h.at[pl.ds(base, SUB)], eb[b], semg)

        def _wait_gathers(b):
            pltpu.make_async_copy(qt_h.at[ixd[b]], qb[b], semg).wait()
            pltpu.make_async_copy(kvt_h.at[ixs[b]], kvb[b], semg).wait()
            pltpu.make_async_copy(ekv_h.at[pl.ds(0, SUB)], eb[b], semg).wait()

        def _wait_scatter(b):
            pltpu.make_async_copy(pbs[b], acc.at[ixc[b]], sems).wait()

        NTP = NSUB // NW
        TAIL = NSUB - NTP * NW

        def _cb(t):
            return (w + t * NW) * SUB

        _load_idx(_cb(0), 0)
        _fire(_cb(0), 0)

        def bigstep(t2, carry):
            for b in range(2):
                t = t2 * 2 + b
                _wait_gathers(b)

                @pl.when(t + 1 < NTP)
                def _():
                    _load_idx(_cb(t + 1), 1 - b)
                    _fire(_cb(t + 1), 1 - b)

                @pl.when(t >= 2)
                def _():
                    _wait_scatter(b)

                _compute(qb[b], kvb[b], eb[b], pbs[b])
                for i in range(SUB // 16):
                    ixc[b][pl.ds(i * 16, 16)] = ixd[b][pl.ds(i * 16, 16)]
                pltpu.async_copy(pbs[b], acc.at[ixc[b]], sems, add=True)
            return carry

        lax.fori_loop(0, NTP // 2, bigstep, 0)
        _wait_scatter(0)
        _wait_scatter(1)

        @pl.when(w < TAIL)
        def _():
            base = (NTP * NW + w) * SUB
            _load_idx(base, 0)
            _fire(base, 0)
            _wait_gathers(0)
            _compute(qb[0], kvb[0], eb[0], pbs[0])
            pltpu.sync_copy(pbs[0], acc.at[ixd[0]], add=True)

        plsc.subcore_barrier()

        @pl.when(s < NS - 1)
        def _():
            pltpu.sync_copy(acc.at[pl.ds(rbase, ROWS_A)],
                            out_h.at[pl.ds(c * NN + rbase, ROWS_A)])

        @pl.when(s == NS - 1)
        def _():
            pltpu.sync_copy(acc.at[pl.ds(rbase, ROWS_B)],
                            out_h.at[pl.ds(c * NN + rbase, ROWS_B)])

    return k(qt, kvt, ekv, dsti, srci)


_BN = 1000
_BE = 2000


def _tc_node(h, wq, wkv):
    def body(h_ref, wq_ref, wkv_ref, q_ref, kv_ref):
        hb = h_ref[...]
        q_ref[...] = jnp.dot(hb, wq_ref[...],
                             preferred_element_type=jnp.float32)
        kv = jnp.dot(hb, wkv_ref[...], preferred_element_type=jnp.float32)
        col = lax.broadcasted_iota(jnp.int32, (_BN, DKV), 1)
        kv_ref[...] = jnp.where(col == DP + DN - 1, 1.0, kv)

    return pl.pallas_call(
        body,
        grid=(NN // _BN,),
        in_specs=[
            pl.BlockSpec((_BN, DN), lambda i: (i, 0)),
            pl.BlockSpec((DN, DP), lambda i: (0, 0)),
            pl.BlockSpec((DN, DKV), lambda i: (0, 0)),
        ],
        out_specs=[
            pl.BlockSpec((_BN, DP), lambda i: (i, 0)),
            pl.BlockSpec((_BN, DKV), lambda i: (i, 0)),
        ],
        out_shape=[
            jax.ShapeDtypeStruct((NN, DP), jnp.float32),
            jax.ShapeDtypeStruct((NN, DKV), jnp.float32),
        ],
    )(h, wq, wkv)


def _tc_geom(pos_src, pos_dst):
    def body(ps_ref, pd_ref, sh_ref, rbf_ref):
        rel = pd_ref[...] - ps_ref[...]
        xx = rel[:, 0:1]
        yy = rel[:, 1:2]
        zz = rel[:, 2:3]
        d = jnp.sqrt(xx * xx + yy * yy + zz * zz)
        inv = 1.0 / (d + 1e-9)
        x = xx * inv
        y = yy * inv
        z = zz * inv
        one = jnp.ones_like(x)
        sh = jnp.concatenate(
            [one, x, y, z,
             x * y, y * z, 0.5 * (3.0 * z * z - 1.0), x * z,
             0.5 * (x * x - y * y),
             y * (3.0 * x * x - y * y), x * y * z, y * (5.0 * z * z - 1.0),
             0.5 * z * (5.0 * z * z - 3.0), x * (5.0 * z * z - 1.0),
             z * (x * x - y * y), x * (x * x - 3.0 * y * y)], axis=1)
        sh_ref[...] = sh
        coli = lax.broadcasted_iota(jnp.int32, (_BE, DSH), 1)
        centers = coli.astype(jnp.float32) * (RAD / (NB - 1))
        rb = jnp.exp(-jnp.square((d - centers) / (RAD / NB)))
        mask = coli < NB
        rbf_ref[...] = jnp.where(mask, rb, 0.0)

    return pl.pallas_call(
        body,
        grid=(NE // _BE,),
        in_specs=[
            pl.BlockSpec((_BE, DP), lambda i: (i, 0)),
            pl.BlockSpec((_BE, DP), lambda i: (i, 0)),
        ],
        out_specs=[
            pl.BlockSpec((_BE, DSH), lambda i: (i, 0)),
            pl.BlockSpec((_BE, DSH), lambda i: (i, 0)),
        ],
        out_shape=[
            jax.ShapeDtypeStruct((NE, DSH), jnp.float32),
            jax.ShapeDtypeStruct((NE, DSH), jnp.float32),
        ],
    )(pos_src, pos_dst)


def _tc_ekv(sh, rbf, skp, svp, r1p, b1, rkp, rvp):
    def body(sh_ref, rbf_ref, sk_ref, sv_ref, r1_ref, b1_ref,
             rk_ref, rv_ref, out_ref):
        shb = sh_ref[...]
        r1 = jnp.maximum(
            jnp.dot(rbf_ref[...], r1_ref[...],
                    preferred_element_type=jnp.float32) + b1_ref[...], 0.0)
        ek = (jnp.dot(shb, sk_ref[...], preferred_element_type=jnp.float32)
              * jnp.dot(r1, rk_ref[...], preferred_element_type=jnp.float32))
        ev = (jnp.dot(shb, sv_ref[...], preferred_element_type=jnp.float32)
              * jnp.dot(r1, rv_ref[...], preferred_element_type=jnp.float32))
        ekv = jnp.concatenate([ek, ev], axis=1)
        col = lax.broadcasted_iota(jnp.int32, (_BE, DKV), 1)
        out_ref[...] = jnp.where(col == DP + DN - 1, 1.0, ekv)

    return pl.pallas_call(
        body,
        grid=(NE // _BE,),
        in_specs=[
            pl.BlockSpec((_BE, DSH), lambda i: (i, 0)),
            pl.BlockSpec((_BE, DSH), lambda i: (i, 0)),
            pl.BlockSpec((DSH, DP), lambda i: (0, 0)),
            pl.BlockSpec((DSH, DP), lambda i: (0, 0)),
            pl.BlockSpec((DSH, HID), lambda i: (0, 0)),
            pl.BlockSpec((1, HID), lambda i: (0, 0)),
            pl.BlockSpec((HID, DP), lambda i: (0, 0)),
            pl.BlockSpec((HID, DP), lambda i: (0, 0)),
        ],
        out_specs=pl.BlockSpec((_BE, DKV), lambda i: (i, 0)),
        out_shape=jax.ShapeDtypeStruct((NE, DKV), jnp.float32),
    )(sh, rbf, skp, svp, r1p, b1, rkp, rvp)


def _tc_finalize(s01, h, wsi, g, skip, dout):
    def body(s0_ref, s1_ref, h_ref, w_ref, g_ref, o_ref):
        num = s0_ref[...] + s1_ref[...]
        col = lax.broadcasted_iota(jnp.int32, (_BN, DN), 1)
        den = jnp.sum(jnp.where(col == (DN - 1), num, 0.0),
                      axis=1, keepdims=True) + 1e-9
        agg = jnp.where(col < dout, num, 0.0) / den
        hb = h_ref[...]
        out = agg + jnp.dot(hb, w_ref[...],
                            preferred_element_type=jnp.float32)
        if skip:
            out = out + hb
        ms = jnp.sum(out * out, axis=1, keepdims=True) / dout
        o_ref[...] = out / jnp.sqrt(ms + 1e-6) * g_ref[...]

    return pl.pallas_call(
        body,
        grid=(NN // _BN,),
        in_specs=[
            pl.BlockSpec((_BN, DN), lambda i: (i, 0)),
            pl.BlockSpec((_BN, DN), lambda i: (i + NN // _BN, 0)),
            pl.BlockSpec((_BN, DN), lambda i: (i, 0)),
            pl.BlockSpec((DN, DN), lambda i: (0, 0)),
            pl.BlockSpec((1, DN), lambda i: (0, 0)),
        ],
        out_specs=pl.BlockSpec((_BN, DN), lambda i: (i, 0)),
        out_shape=jax.ShapeDtypeStruct((NN, DN), jnp.float32),
    )(s01, s01, h, wsi, g)


def _pad2(a, r, c):
    return jnp.zeros((r, c), jnp.float32).at[:a.shape[0], :a.shape[1]].set(a)


def kernel(x, pos, params, edge_index):
    src = edge_index[0].astype(jnp.int32)
    dst = edge_index[1].astype(jnp.int32)

    pos_pad = _pad2(pos, NN, DP)
    x_pad = _pad2(x, NN, DN)

    pos_d, pos_s = _sc_gather_multi([pos_pad, pos_pad], [dst, src])
    sh, rbf = _tc_geom(pos_s, pos_d)

    h = x_pad
    douts = [86, 86, 86, 86, 86, 86, 40]
    skips = [False, True, True, True, True, True, False]
    for p, dout, skip in zip(params, douts, skips):
        din = p['Wq'].shape[0]
        wq = _pad2(p['Wq'], DN, DP)
        wkv = jnp.zeros((DN, DKV), jnp.float32)
        wkv = wkv.at[:din, :ADIM].set(p['Wk'])
        wkv = wkv.at[:din, DP:DP + dout].set(p['Wv'])
        skp = _pad2(p['Sk'], DSH, DP)
        svp = _pad2(p['Sv'], DSH, DP)
        r1p = _pad2(p['R1'], DSH, HID)
        b1 = p['b1'].reshape(1, HID)
        rkp = _pad2(p['Rk'], HID, DP)
        rvp = _pad2(p['Rv'], HID, DP)
        wsi = _pad2(p['Wsi'], DN, DN)
        g = _pad2(p['g'].reshape(1, -1), 1, DN)

        qt, kvt = _tc_node(h, wq, wkv)
        ekv = _tc_ekv(sh, rbf, skp, svp, r1p, b1, rkp, rvp)
        s01 = _sc_edge_pass(qt, kvt, ekv, dst, src)
        h = _tc_finalize(s01, h, wsi, g, skip, dout)

    return h[:, :40]


# speedup vs baseline: 1.4527x; 1.1049x over previous
import functools

import jax
import jax.numpy as jnp
import numpy as np
from jax import lax
from jax.experimental import pallas as pl
from jax.experimental.pallas import tpu as pltpu
from jax.experimental.pallas import tpu_sc as plsc

NN = 10000
NE = 160000
DP = 128
DKV = 2 * DP
DN = 128
DSH = 16
ADIM = 86
RAD = 5.0
NB = 10
HID = 32
INV_SQRT_ATT = 1.0 / np.sqrt(ADIM)

NC, NS = 2, 16
NW = NC * NS
SUB = 32
NSUB = NE // SUB
NT_BASE = NSUB // NW
NT_REM = NSUB - NT_BASE * NW
ROWS_A = 624
ROWS_B = NN - (NS - 1) * ROWS_A
CGRP = 88

_mesh = functools.partial(
    plsc.VectorSubcoreMesh, core_axis_name="c", subcore_axis_name="s")


def _sc_gather_multi(tables, idxs):
    ntab = len(tables)
    widths = [t.shape[1] for t in tables]

    @functools.partial(
        pl.kernel,
        mesh=_mesh(),
        out_type=[jax.ShapeDtypeStruct((NE, w), jnp.float32)
                  for w in widths],
        scratch_types=(
            [pltpu.VMEM((SUB,), jnp.int32) for _ in range(ntab)]
            + [pltpu.VMEM((SUB, w), jnp.float32) for w in widths]
            + [pltpu.SemaphoreType.DMA]
        ),
    )
    def k(*refs):
        tab_h = refs[:ntab]
        idx_h = refs[ntab:2 * ntab]
        out_h = refs[2 * ntab:3 * ntab]
        ix = refs[3 * ntab:4 * ntab]
        rr = refs[4 * ntab:5 * ntab]
        sem = refs[5 * ntab]
        w = lax.axis_index("s") * NC + lax.axis_index("c")
        nt = NT_BASE + jnp.where(w < NT_REM, 1, 0)

        def body(t, carry):
            j = w + t * NW
            base = j * SUB
            for n in range(ntab):
                pltpu.sync_copy(idx_h[n].at[pl.ds(base, SUB)], ix[n])
            cps = [pltpu.async_copy(tab_h[n].at[ix[n]], rr[n], sem)
                   for n in range(ntab)]
            for cp in cps:
                cp.wait()
            for n in range(ntab):
                pltpu.sync_copy(rr[n], out_h[n].at[pl.ds(base, SUB)])
            return carry

        lax.fori_loop(0, nt, body, 0)

    return k(*tables, *idxs)


def _sc_edge_pass(qt, kvt, ekv, dsti):
    @functools.partial(
        pl.kernel,
        mesh=_mesh(),
        out_type=jax.ShapeDtypeStruct((NC * NN, DN), jnp.float32),
        scratch_types=[
            pltpu.VMEM((2 * SUB,), jnp.int32),
            pltpu.VMEM((2 * SUB,), jnp.int32),
            pltpu.VMEM((SUB,), jnp.int32),
            pltpu.VMEM((SUB,), jnp.int32),
            pltpu.VMEM((SUB, DP), jnp.float32),
            pltpu.VMEM((SUB, DP), jnp.float32),
            pltpu.VMEM((SUB, DKV), jnp.float32),
            pltpu.VMEM((SUB, DKV), jnp.float32),
            pltpu.VMEM((SUB, DKV), jnp.float32),
            pltpu.VMEM((SUB, DKV), jnp.float32),
            pltpu.VMEM((SUB, DN), jnp.float32),
            pltpu.VMEM((SUB, DN), jnp.float32),
            pltpu.VMEM_SHARED((NN, DN), jnp.float32),
            pltpu.SemaphoreType.DMA,
            pltpu.SemaphoreType.DMA,
        ],
    )
    def k(qt_h, kvt_h, ekv_h, ds_h, out_h,
          ix0, ix1, ixc0, ixc1,
          qb0, qb1, kvb0, kvb1, eb0, eb1, pb0, pb1,
          acc, semg, sems):
        ixp = (ix0, ix1)
        ixc = (ixc0, ixc1)
        qb = (qb0, qb1)
        kvb = (kvb0, kvb1)
        eb = (eb0, eb1)
        pbs = (pb0, pb1)
        pb = pb0
        c = lax.axis_index("c")
        s = lax.axis_index("s")
        w = s * NC + c
        rbase = s * ROWS_A

        def zrow(e, cz):
            for i in range(DN // 16):
                pb[e, pl.ds(i * 16, 16)] = jnp.zeros((16,), jnp.float32)
            return cz

        lax.fori_loop(0, SUB, zrow, 0)

        def zcp(i, cz):
            pltpu.sync_copy(pb, acc.at[pl.ds(rbase + i * SUB, SUB)])
            return cz

        lax.fori_loop(0, ROWS_A // SUB, zcp, 0)

        @pl.when(s < NS - 1)
        def _():
            pltpu.sync_copy(pb.at[pl.ds(0, ROWS_A - (ROWS_A // SUB) * SUB)],
                            acc.at[pl.ds(rbase + (ROWS_A // SUB) * SUB,
                                         ROWS_A - (ROWS_A // SUB) * SUB)])

        @pl.when(s == NS - 1)
        def _():
            pltpu.sync_copy(pb.at[pl.ds(0, ROWS_B - (ROWS_A // SUB) * SUB)],
                            acc.at[pl.ds(rbase + (ROWS_A // SUB) * SUB,
                                         ROWS_B - (ROWS_A // SUB) * SUB)])

        plsc.subcore_barrier()
        lane = jnp.arange(16, dtype=jnp.int32)

        def _hsum(a):
            for shf in (8, 4, 2, 1):
                a = a + a.at[lane ^ shf].get(mode="promise_in_bounds")
            return a

        def _compute(q_b, kv_b, e_b, p_b):
            def edge(e, c1):
                a = jnp.zeros((16,), jnp.float32)
                for i in range(6):
                    qv = q_b[e, pl.ds(i * 16, 16)]
                    kv = kv_b[e, pl.ds(i * 16, 16)]
                    ev = e_b[e, pl.ds(i * 16, 16)]
                    a = a + qv * kv * ev
                exv = jnp.exp(_hsum(a) * INV_SQRT_ATT)
                for i in range(DN // 16):
                    vv = kv_b[e, pl.ds(DP + i * 16, 16)]
                    evv = e_b[e, pl.ds(DP + i * 16, 16)]
                    p_b[e, pl.ds(i * 16, 16)] = exv * vv * evv
                return c1

            lax.fori_loop(0, SUB, edge, 0)

        def _load_idx(base, b):
            pltpu.sync_copy(ds_h.at[pl.ds(2 * base, 2 * SUB)], ixp[b])

        def _fire(base, b):
            pltpu.async_copy(qt_h.at[ixp[b].at[pl.ds(0, SUB)]], qb[b], semg)
            pltpu.async_copy(kvt_h.at[ixp[b].at[pl.ds(SUB, SUB)]], kvb[b],
                             semg)
            pltpu.async_copy(ekv_h.at[pl.ds(base, SUB)], eb[b], semg)

        def _wait_gathers(b):
            pltpu.make_async_copy(qt_h.at[ixp[b].at[pl.ds(0, SUB)]],
                                  qb[b], semg).wait()
            pltpu.make_async_copy(kvt_h.at[ixp[b].at[pl.ds(SUB, SUB)]],
                                  kvb[b], semg).wait()
            pltpu.make_async_copy(ekv_h.at[pl.ds(0, SUB)], eb[b], semg).wait()

        def _wait_scatter(b):
            pltpu.make_async_copy(pbs[b], acc.at[ixc[b]], sems).wait()

        NTP = NSUB // NW
        TAIL = NSUB - NTP * NW

        def _cb(t):
            return (w + t * NW) * SUB

        _load_idx(_cb(0), 0)
        _fire(_cb(0), 0)

        def bigstep(t2, carry):
            for b in range(2):
                t = t2 * 2 + b
                _wait_gathers(b)

                @pl.when(t + 1 < NTP)
                def _():
                    _load_idx(_cb(t + 1), 1 - b)
                    _fire(_cb(t + 1), 1 - b)

                @pl.when(t >= 2)
                def _():
                    _wait_scatter(b)

                _compute(qb[b], kvb[b], eb[b], pbs[b])
                for i in range(SUB // 16):
                    ixc[b][pl.ds(i * 16, 16)] = ixp[b][pl.ds(i * 16, 16)]
                pltpu.async_copy(pbs[b], acc.at[ixc[b]], sems, add=True)
            return carry

        lax.fori_loop(0, NTP // 2, bigstep, 0)
        _wait_scatter(0)
        _wait_scatter(1)

        @pl.when(w < TAIL)
        def _():
            base = (NTP * NW + w) * SUB
            _load_idx(base, 0)
            _fire(base, 0)
            _wait_gathers(0)
            _compute(qb[0], kvb[0], eb[0], pbs[0])
            for i in range(SUB // 16):
                ixc[0][pl.ds(i * 16, 16)] = ixp[0][pl.ds(i * 16, 16)]
            pltpu.sync_copy(pbs[0], acc.at[ixc[0]], add=True)

        plsc.subcore_barrier()

        @pl.when(s < NS - 1)
        def _():
            pltpu.sync_copy(acc.at[pl.ds(rbase, ROWS_A)],
                            out_h.at[pl.ds(c * NN + rbase, ROWS_A)])

        @pl.when(s == NS - 1)
        def _():
            pltpu.sync_copy(acc.at[pl.ds(rbase, ROWS_B)],
                            out_h.at[pl.ds(c * NN + rbase, ROWS_B)])

    return k(qt, kvt, ekv, dsti)


_BN = 1000
_BE = 2000


def _tc_node(h, wq, wkv):
    def body(h_ref, wq_ref, wkv_ref, q_ref, kv_ref):
        hb = h_ref[...]
        q_ref[...] = jnp.dot(hb, wq_ref[...],
                             preferred_element_type=jnp.float32)
        kv = jnp.dot(hb, wkv_ref[...], preferred_element_type=jnp.float32)
        col = lax.broadcasted_iota(jnp.int32, (_BN, DKV), 1)
        kv_ref[...] = jnp.where(col == DP + DN - 1, 1.0, kv)

    return pl.pallas_call(
        body,
        grid=(NN // _BN,),
        in_specs=[
            pl.BlockSpec((_BN, DN), lambda i: (i, 0)),
            pl.BlockSpec((DN, DP), lambda i: (0, 0)),
            pl.BlockSpec((DN, DKV), lambda i: (0, 0)),
        ],
        out_specs=[
            pl.BlockSpec((_BN, DP), lambda i: (i, 0)),
            pl.BlockSpec((_BN, DKV), lambda i: (i, 0)),
        ],
        out_shape=[
            jax.ShapeDtypeStruct((NN, DP), jnp.float32),
            jax.ShapeDtypeStruct((NN, DKV), jnp.float32),
        ],
    )(h, wq, wkv)


def _tc_geom(pos_src, pos_dst):
    def body(ps_ref, pd_ref, sh_ref, rbf_ref):
        rel = pd_ref[...] - ps_ref[...]
        xx = rel[:, 0:1]
        yy = rel[:, 1:2]
        zz = rel[:, 2:3]
        d = jnp.sqrt(xx * xx + yy * yy + zz * zz)
        inv = 1.0 / (d + 1e-9)
        x = xx * inv
        y = yy * inv
        z = zz * inv
        one = jnp.ones_like(x)
        sh = jnp.concatenate(
            [one, x, y, z,
             x * y, y * z, 0.5 * (3.0 * z * z - 1.0), x * z,
             0.5 * (x * x - y * y),
             y * (3.0 * x * x - y * y), x * y * z, y * (5.0 * z * z - 1.0),
             0.5 * z * (5.0 * z * z - 3.0), x * (5.0 * z * z - 1.0),
             z * (x * x - y * y), x * (x * x - 3.0 * y * y)], axis=1)
        sh_ref[...] = sh
        coli = lax.broadcasted_iota(jnp.int32, (_BE, DSH), 1)
        centers = coli.astype(jnp.float32) * (RAD / (NB - 1))
        rb = jnp.exp(-jnp.square((d - centers) / (RAD / NB)))
        mask = coli < NB
        rbf_ref[...] = jnp.where(mask, rb, 0.0)

    return pl.pallas_call(
        body,
        grid=(NE // _BE,),
        in_specs=[
            pl.BlockSpec((_BE, DP), lambda i: (i, 0)),
            pl.BlockSpec((_BE, DP), lambda i: (i, 0)),
        ],
        out_specs=[
            pl.BlockSpec((_BE, DSH), lambda i: (i, 0)),
            pl.BlockSpec((_BE, DSH), lambda i: (i, 0)),
        ],
        out_shape=[
            jax.ShapeDtypeStruct((NE, DSH), jnp.float32),
            jax.ShapeDtypeStruct((NE, DSH), jnp.float32),
        ],
    )(pos_src, pos_dst)


def _tc_ekv(sh, rbf, skp, svp, r1p, b1, rkp, rvp):
    def body(sh_ref, rbf_ref, sk_ref, sv_ref, r1_ref, b1_ref,
             rk_ref, rv_ref, out_ref):
        shb = sh_ref[...]
        r1 = jnp.maximum(
            jnp.dot(rbf_ref[...], r1_ref[...],
                    preferred_element_type=jnp.float32) + b1_ref[...], 0.0)
        ek = (jnp.dot(shb, sk_ref[...], preferred_element_type=jnp.float32)
              * jnp.dot(r1, rk_ref[...], preferred_element_type=jnp.float32))
        ev = (jnp.dot(shb, sv_ref[...], preferred_element_type=jnp.float32)
              * jnp.dot(r1, rv_ref[...], preferred_element_type=jnp.float32))
        ekv = jnp.concatenate([ek, ev], axis=1)
        col = lax.broadcasted_iota(jnp.int32, (_BE, DKV), 1)
        out_ref[...] = jnp.where(col == DP + DN - 1, 1.0, ekv)

    return pl.pallas_call(
        body,
        grid=(NE // _BE,),
        in_specs=[
            pl.BlockSpec((_BE, DSH), lambda i: (i, 0)),
            pl.BlockSpec((_BE, DSH), lambda i: (i, 0)),
            pl.BlockSpec((DSH, DP), lambda i: (0, 0)),
            pl.BlockSpec((DSH, DP), lambda i: (0, 0)),
            pl.BlockSpec((DSH, HID), lambda i: (0, 0)),
            pl.BlockSpec((1, HID), lambda i: (0, 0)),
            pl.BlockSpec((HID, DP), lambda i: (0, 0)),
            pl.BlockSpec((HID, DP), lambda i: (0, 0)),
        ],
        out_specs=pl.BlockSpec((_BE, DKV), lambda i: (i, 0)),
        out_shape=jax.ShapeDtypeStruct((NE, DKV), jnp.float32),
    )(sh, rbf, skp, svp, r1p, b1, rkp, rvp)


def _finalize_math(s0, s1, hb, wsi, gg, skip, dout):
    num = s0 + s1
    col = lax.broadcasted_iota(jnp.int32, (_BN, DN), 1)
    den = jnp.sum(jnp.where(col == (DN - 1), num, 0.0),
                  axis=1, keepdims=True) + 1e-9
    agg = jnp.where(col < dout, num, 0.0) / den
    out = agg + jnp.dot(hb, wsi, preferred_element_type=jnp.float32)
    if skip:
        out = out + hb
    ms = jnp.sum(out * out, axis=1, keepdims=True) / dout
    return out / jnp.sqrt(ms + 1e-6) * gg


def _tc_finalize(s01, h, wsi, g, skip, dout):
    def body(s0_ref, s1_ref, h_ref, w_ref, g_ref, o_ref):
        o_ref[...] = _finalize_math(s0_ref[...], s1_ref[...], h_ref[...],
                                    w_ref[...], g_ref[...], skip, dout)

    return pl.pallas_call(
        body,
        grid=(NN // _BN,),
        in_specs=[
            pl.BlockSpec((_BN, DN), lambda i: (i, 0)),
            pl.BlockSpec((_BN, DN), lambda i: (i + NN // _BN, 0)),
            pl.BlockSpec((_BN, DN), lambda i: (i, 0)),
            pl.BlockSpec((DN, DN), lambda i: (0, 0)),
            pl.BlockSpec((1, DN), lambda i: (0, 0)),
        ],
        out_specs=pl.BlockSpec((_BN, DN), lambda i: (i, 0)),
        out_shape=jax.ShapeDtypeStruct((NN, DN), jnp.float32),
    )(s01, s01, h, wsi, g)


def _tc_finalize_node(s01, h, wsi, g, skip, dout, wq, wkv):
    def body(s0_ref, s1_ref, h_ref, w_ref, g_ref, wq_ref, wkv_ref,
             o_ref, q_ref, kv_ref):
        hn = _finalize_math(s0_ref[...], s1_ref[...], h_ref[...],
                            w_ref[...], g_ref[...], skip, dout)
        o_ref[...] = hn
        q_ref[...] = jnp.dot(hn, wq_ref[...],
                             preferred_element_type=jnp.float32)
        kv = jnp.dot(hn, wkv_ref[...], preferred_element_type=jnp.float32)
        col = lax.broadcasted_iota(jnp.int32, (_BN, DKV), 1)
        kv_ref[...] = jnp.where(col == DP + DN - 1, 1.0, kv)

    return pl.pallas_call(
        body,
        grid=(NN // _BN,),
        in_specs=[
            pl.BlockSpec((_BN, DN), lambda i: (i, 0)),
            pl.BlockSpec((_BN, DN), lambda i: (i + NN // _BN, 0)),
            pl.BlockSpec((_BN, DN), lambda i: (i, 0)),
            pl.BlockSpec((DN, DN), lambda i: (0, 0)),
            pl.BlockSpec((1, DN), lambda i: (0, 0)),
            pl.BlockSpec((DN, DP), lambda i: (0, 0)),
            pl.BlockSpec((DN, DKV), lambda i: (0, 0)),
        ],
        out_specs=[
            pl.BlockSpec((_BN, DN), lambda i: (i, 0)),
            pl.BlockSpec((_BN, DP), lambda i: (i, 0)),
            pl.BlockSpec((_BN, DKV), lambda i: (i, 0)),
        ],
        out_shape=[
            jax.ShapeDtypeStruct((NN, DN), jnp.float32),
            jax.ShapeDtypeStruct((NN, DP), jnp.float32),
            jax.ShapeDtypeStruct((NN, DKV), jnp.float32),
        ],
    )(s01, s01, h, wsi, g, wq, wkv)


def _pad2(a, r, c):
    return jnp.zeros((r, c), jnp.float32).at[:a.shape[0], :a.shape[1]].set(a)


def kernel(x, pos, params, edge_index):
    src = edge_index[0].astype(jnp.int32)
    dst = edge_index[1].astype(jnp.int32)

    pos_pad = _pad2(pos, NN, DP)
    x_pad = _pad2(x, NN, DN)

    ds_packed = jnp.stack([dst.reshape(NSUB, SUB), src.reshape(NSUB, SUB)],
                          axis=1).reshape(-1)

    pos_d, pos_s = _sc_gather_multi([pos_pad, pos_pad], [dst, src])
    sh, rbf = _tc_geom(pos_s, pos_d)

    h = x_pad
    douts = [86, 86, 86, 86, 86, 86, 40]
    skips = [False, True, True, True, True, True, False]
    pp = []
    for p, dout in zip(params, douts):
        din = p['Wq'].shape[0]
        wkv = jnp.zeros((DN, DKV), jnp.float32)
        wkv = wkv.at[:din, :ADIM].set(p['Wk'])
        wkv = wkv.at[:din, DP:DP + dout].set(p['Wv'])
        pp.append(dict(
            wq=_pad2(p['Wq'], DN, DP), wkv=wkv,
            skp=_pad2(p['Sk'], DSH, DP), svp=_pad2(p['Sv'], DSH, DP),
            r1p=_pad2(p['R1'], DSH, HID), b1=p['b1'].reshape(1, HID),
            rkp=_pad2(p['Rk'], HID, DP), rvp=_pad2(p['Rv'], HID, DP),
            wsi=_pad2(p['Wsi'], DN, DN),
            g=_pad2(p['g'].reshape(1, -1), 1, DN)))

    qt, kvt = _tc_node(h, pp[0]['wq'], pp[0]['wkv'])
    for li, (q, dout, skip) in enumerate(zip(pp, douts, skips)):
        ekv = _tc_ekv(sh, rbf, q['skp'], q['svp'], q['r1p'], q['b1'],
                      q['rkp'], q['rvp'])
        s01 = _sc_edge_pass(qt, kvt, ekv, ds_packed)
        if li < 6:
            h, qt, kvt = _tc_finalize_node(
                s01, h, q['wsi'], q['g'], skip, dout,
                pp[li + 1]['wq'], pp[li + 1]['wkv'])
        else:
            h = _tc_finalize(s01, h, q['wsi'], q['g'], skip, dout)

    return h[:, :40]
